# baseline (speedup 1.0000x reference)
import functools

import jax
import jax.numpy as jnp
from jax import lax
from jax.experimental import pallas as pl
from jax.experimental.pallas import tpu as pltpu
from jax.experimental.pallas import tpu_sc as plsc

R, C = 64, 100000
BC = 2048
GRID = (C + BC - 1) // BC


def _rotl(x, r):
    return (x << jnp.uint32(r)) | (x >> jnp.uint32(32 - r))


def _threefry_bits(n):
    k0 = jnp.uint32(0)
    k1 = jnp.uint32(42)
    ks = [k0, k1, jnp.uint32(0x1BD11BDA) ^ k0 ^ k1]
    rot_even = (13, 15, 26, 6)
    rot_odd = (17, 29, 16, 24)
    t = n + k1
    x0 = t
    x1 = _rotl(t, 13) ^ t
    for r in rot_even[1:]:
        x0 = x0 + x1
        x1 = _rotl(x1, r)
        x1 = x1 ^ x0
    x0 = x0 + ks[1]
    x1 = x1 + ks[2] + jnp.uint32(1)
    for i in range(1, 5):
        for r in rot_even if i % 2 == 0 else rot_odd:
            x0 = x0 + x1
            x1 = _rotl(x1, r)
            x1 = x1 ^ x0
        x0 = x0 + ks[(i + 1) % 3]
        x1 = x1 + ks[(i + 2) % 3] + jnp.uint32(i + 1)
    return x0 ^ x1


def _gumbel(rows, cols):
    n = (rows * C + cols).astype(jnp.uint32)
    bits = _threefry_bits(n)
    fb = (bits >> jnp.uint32(9)) | jnp.uint32(0x3F800000)
    u = jax.lax.bitcast_convert_type(fb, jnp.float32) - jnp.float32(1.0)
    eps = jnp.float32(1e-10)
    return -jnp.log(-jnp.log(u + eps) + eps)


def _argmax_body(x_ref, o_ref, maxv_ref, argf_ref):
    step = pl.program_id(0)
    shape = (R, BC)
    cols = jax.lax.broadcasted_iota(jnp.int32, shape, 1) + step * BC
    rows = jax.lax.broadcasted_iota(jnp.int32, shape, 0)
    y = x_ref[...] + _gumbel(rows, cols)

    def reduce_and_merge(y, cols):
        m = jnp.max(y, axis=1, keepdims=True)
        cand = jnp.where(y == m, cols, jnp.int32(2**31 - 1))
        a = jnp.min(cand, axis=1, keepdims=True)

        @pl.when(step == 0)
        def _():
            maxv_ref[...] = m
            argf_ref[...] = a

        @pl.when(step > 0)
        def _():
            upd = m > maxv_ref[...]
            maxv_ref[...] = jnp.where(upd, m, maxv_ref[...])
            argf_ref[...] = jnp.where(upd, a, argf_ref[...])

    @pl.when(step < GRID - 1)
    def _():
        reduce_and_merge(y, cols)
        o_ref[...] = jnp.zeros(shape, jnp.float32)

    @pl.when(step == GRID - 1)
    def _():
        reduce_and_merge(jnp.where(cols < C, y, -jnp.inf), cols)
        o_ref[...] = (cols == argf_ref[...]).astype(jnp.float32)


_sc_mesh = plsc.VectorSubcoreMesh(core_axis_name="c", subcore_axis_name="s")


@functools.partial(
    pl.kernel,
    mesh=_sc_mesh,
    scratch_types=[
        pltpu.VMEM((R,), jnp.int32),
        pltpu.VMEM((R,), jnp.float32),
        pltpu.SemaphoreType.DMA,
    ],
)
def _sc_scatter_ones(idx_hbm, out_hbm, idx_v, ones_v, sem):
    @pl.when((lax.axis_index("c") == 0) & (lax.axis_index("s") == 0))
    def _():
        pltpu.sync_copy(idx_hbm, idx_v)
        for i in range(R // 16):
            ones_v[pl.ds(i * 16, 16)] = jnp.ones((16,), jnp.float32)
        pltpu.async_copy(ones_v, out_hbm.at[idx_v], sem).wait()


@jax.jit
def kernel(logits):
    zeros_out, _, argf = pl.pallas_call(
        _argmax_body,
        grid=(GRID,),
        in_specs=[pl.BlockSpec((R, BC), lambda i: (0, i))],
        out_specs=[
            pl.BlockSpec((R, BC), lambda i: (0, i)),
            pl.BlockSpec((R, 1), lambda i: (0, 0)),
            pl.BlockSpec((R, 1), lambda i: (0, 0)),
        ],
        out_shape=[
            jax.ShapeDtypeStruct((R, C), jnp.float32),
            jax.ShapeDtypeStruct((R, 1), jnp.float32),
            jax.ShapeDtypeStruct((R, 1), jnp.int32),
        ],
        compiler_params=pltpu.CompilerParams(
            dimension_semantics=("arbitrary",)),
    )(logits)
    return _patch_ones(argf, zeros_out)


_LAST_BASE = (GRID - 1) * BC
_MAX_WIN = _LAST_BASE - 128


def _patch_body(argc_v_ref, argc_s_ref, z_ref, o_ref, pat_ref, sem):
    copies = []
    for r in range(R):
        g = r // 8
        base = jnp.minimum((argc_s_ref[r, 0] // 128) * 128, _MAX_WIN)
        argc_g = argc_v_ref[pl.ds(8 * g, 8), :]
        lanes = jax.lax.broadcasted_iota(jnp.int32, (8, 128), 1) + base
        pat_ref[r] = (argc_g == lanes).astype(jnp.float32)
        cp = pltpu.make_async_copy(
            pat_ref.at[r],
            o_ref.at[pl.ds(8 * g, 8), pl.ds(base, 128)],
            sem)
        cp.start()
        copies.append(cp)
    for cp in copies:
        cp.wait()


def _patch_ones(argc, zeros_out):
    return pl.pallas_call(
        _patch_body,
        in_specs=[
            pl.BlockSpec(memory_space=pltpu.VMEM),
            pl.BlockSpec(memory_space=pltpu.SMEM),
            pl.BlockSpec(memory_space=pl.ANY),
        ],
        out_specs=pl.BlockSpec(memory_space=pl.ANY),
        out_shape=jax.ShapeDtypeStruct((R, C), jnp.float32),
        scratch_shapes=[
            pltpu.VMEM((R, 8, 128), jnp.float32),
            pltpu.SemaphoreType.DMA,
        ],
        input_output_aliases={2: 0},
    )(argc, argc, zeros_out)


# speedup vs baseline: 1.5233x; 1.5233x over previous
import functools

import jax
import jax.numpy as jnp
from jax import lax
from jax.experimental import pallas as pl
from jax.experimental.pallas import tpu as pltpu
from jax.experimental.pallas import tpu_sc as plsc

R, C = 64, 100000
BC = 2048
GRID = (C + BC - 1) // BC


def _rotl(x, r):
    return (x << jnp.uint32(r)) | (x >> jnp.uint32(32 - r))


def _threefry_bits(n):
    k0 = jnp.uint32(0)
    k1 = jnp.uint32(42)
    ks = [k0, k1, jnp.uint32(0x1BD11BDA) ^ k0 ^ k1]
    rot_even = (13, 15, 26, 6)
    rot_odd = (17, 29, 16, 24)
    t = n + k1
    x0 = t
    x1 = _rotl(t, 13) ^ t
    for r in rot_even[1:]:
        x0 = x0 + x1
        x1 = _rotl(x1, r)
        x1 = x1 ^ x0
    x0 = x0 + ks[1]
    x1 = x1 + ks[2] + jnp.uint32(1)
    for i in range(1, 5):
        for r in rot_even if i % 2 == 0 else rot_odd:
            x0 = x0 + x1
            x1 = _rotl(x1, r)
            x1 = x1 ^ x0
        x0 = x0 + ks[(i + 1) % 3]
        x1 = x1 + ks[(i + 2) % 3] + jnp.uint32(i + 1)
    return x0 ^ x1


def _gumbel(rows, cols):
    n = (rows * C + cols).astype(jnp.uint32)
    bits = _threefry_bits(n)
    fb = (bits >> jnp.uint32(9)) | jnp.uint32(0x3F800000)
    u = jax.lax.bitcast_convert_type(fb, jnp.float32) - jnp.float32(1.0)
    eps = jnp.float32(1e-10)
    return -jnp.log(-jnp.log(u + eps) + eps)


def _argmax_body(x_ref, o_ref, maxv_ref, argf_ref):
    step = pl.program_id(0)
    shape = (R, BC)
    cols = jax.lax.broadcasted_iota(jnp.int32, shape, 1) + step * BC
    rows = jax.lax.broadcasted_iota(jnp.int32, shape, 0)
    y = x_ref[...] + _gumbel(rows, cols)
    y = jnp.where(cols < C, y, -jnp.inf)
    m = jnp.max(y, axis=1, keepdims=True)
    cand = jnp.where(y == m, cols, jnp.int32(2**31 - 1))
    a = jnp.min(cand, axis=1, keepdims=True)

    @pl.when(step == 0)
    def _():
        maxv_ref[...] = m
        argf_ref[...] = a

    @pl.when(step > 0)
    def _():
        upd = m > maxv_ref[...]
        maxv_ref[...] = jnp.where(upd, m, maxv_ref[...])
        argf_ref[...] = jnp.where(upd, a, argf_ref[...])

    @pl.when(step < GRID - 1)
    def _():
        o_ref[...] = jnp.zeros(shape, jnp.float32)

    @pl.when(step == GRID - 1)
    def _():
        o_ref[...] = (cols == argf_ref[...]).astype(jnp.float32)


_sc_mesh = plsc.VectorSubcoreMesh(core_axis_name="c", subcore_axis_name="s")


@functools.partial(
    pl.kernel,
    mesh=_sc_mesh,
    scratch_types=[
        pltpu.VMEM((R,), jnp.int32),
        pltpu.VMEM((R,), jnp.float32),
        pltpu.SemaphoreType.DMA,
    ],
)
def _sc_scatter_ones(idx_hbm, out_hbm, idx_v, ones_v, sem):
    @pl.when((lax.axis_index("c") == 0) & (lax.axis_index("s") == 0))
    def _():
        pltpu.sync_copy(idx_hbm, idx_v)
        for i in range(R // 16):
            ones_v[pl.ds(i * 16, 16)] = jnp.ones((16,), jnp.float32)
        pltpu.async_copy(ones_v, out_hbm.at[idx_v], sem).wait()


@jax.jit
def kernel(logits):
    zeros_out, _, argf = pl.pallas_call(
        _argmax_body,
        grid=(GRID,),
        in_specs=[pl.BlockSpec((R, BC), lambda i: (0, i))],
        out_specs=[
            pl.BlockSpec((R, BC), lambda i: (0, i)),
            pl.BlockSpec((R, 1), lambda i: (0, 0)),
            pl.BlockSpec((R, 1), lambda i: (0, 0)),
        ],
        out_shape=[
            jax.ShapeDtypeStruct((R, C), jnp.float32),
            jax.ShapeDtypeStruct((R, 1), jnp.float32),
            jax.ShapeDtypeStruct((R, 1), jnp.int32),
        ],
        compiler_params=pltpu.CompilerParams(
            dimension_semantics=("arbitrary",)),
    )(logits)
    return _patch_ones(argf, zeros_out)


_LAST_BASE = (GRID - 1) * BC
_MAX_WIN = _LAST_BASE - 128


def _patch_body(argc_v_ref, argc_s_ref, z_ref, o_ref, pat_ref, sem):
    copies = []
    for r in range(R):
        g = r // 8
        base = jnp.minimum((argc_s_ref[r, 0] // 128) * 128, _MAX_WIN)
        argc_g = argc_v_ref[pl.ds(8 * g, 8), :]
        lanes = jax.lax.broadcasted_iota(jnp.int32, (8, 128), 1) + base
        pat_ref[r] = (argc_g == lanes).astype(jnp.float32)
        cp = pltpu.make_async_copy(
            pat_ref.at[r],
            o_ref.at[pl.ds(8 * g, 8), pl.ds(base, 128)],
            sem)
        cp.start()
        copies.append(cp)
    for cp in copies:
        cp.wait()


def _patch_ones(argc, zeros_out):
    return pl.pallas_call(
        _patch_body,
        in_specs=[
            pl.BlockSpec(memory_space=pltpu.VMEM),
            pl.BlockSpec(memory_space=pltpu.SMEM),
            pl.BlockSpec(memory_space=pl.ANY),
        ],
        out_specs=pl.BlockSpec(memory_space=pl.ANY),
        out_shape=jax.ShapeDtypeStruct((R, C), jnp.float32),
        scratch_shapes=[
            pltpu.VMEM((R, 8, 128), jnp.float32),
            pltpu.SemaphoreType.DMA,
        ],
        input_output_aliases={2: 0},
    )(argc, argc, zeros_out)


# speedup vs baseline: 1.6934x; 1.1117x over previous
import jax
import jax.numpy as jnp
from jax.experimental import pallas as pl
from jax.experimental.pallas import tpu as pltpu

R, C = 64, 100000
BC = 2048
GRID = (C + BC - 1) // BC
FW = 256


def _rotl(x, r):
    return (x << jnp.uint32(r)) | (x >> jnp.uint32(32 - r))


def _threefry_bits(n):
    k0 = jnp.uint32(0)
    k1 = jnp.uint32(42)
    ks = [k0, k1, jnp.uint32(0x1BD11BDA) ^ k0 ^ k1]
    rot_even = (13, 15, 26, 6)
    rot_odd = (17, 29, 16, 24)
    t = n + k1
    x0 = t
    x1 = _rotl(t, 13) ^ t
    for r in rot_even[1:]:
        x0 = x0 + x1
        x1 = _rotl(x1, r)
        x1 = x1 ^ x0
    x0 = x0 + ks[1]
    x1 = x1 + ks[2] + jnp.uint32(1)
    for i in range(1, 5):
        for r in rot_even if i % 2 == 0 else rot_odd:
            x0 = x0 + x1
            x1 = _rotl(x1, r)
            x1 = x1 ^ x0
        x0 = x0 + ks[(i + 1) % 3]
        x1 = x1 + ks[(i + 2) % 3] + jnp.uint32(i + 1)
    return x0 ^ x1


def _gumbel(n):
    bits = _threefry_bits(n.astype(jnp.uint32))
    fb = (bits >> jnp.uint32(9)) | jnp.uint32(0x3F800000)
    u = jax.lax.bitcast_convert_type(fb, jnp.float32) - jnp.float32(1.0)
    eps = jnp.float32(1e-10)
    return -jnp.log(-jnp.log(u + eps) + eps)


def _argmax_body(x_ref, o_ref, argf_ref, n_ref, acc_ref, argn_ref):
    step = pl.program_id(0)
    shape = (R, BC)

    @pl.when(step == 0)
    def _():
        cols = jax.lax.broadcasted_iota(jnp.int32, shape, 1)
        rows = jax.lax.broadcasted_iota(jnp.int32, shape, 0)
        n_ref[...] = rows * C + cols
        acc_ref[...] = jnp.full((R, FW), -jnp.inf, jnp.float32)
        argn_ref[...] = jnp.zeros((R, FW), jnp.int32)

    n = n_ref[...]
    y = x_ref[...] + _gumbel(n)
    n_ref[...] = n + BC

    rowlim = (jax.lax.broadcasted_iota(jnp.int32, (R, 1), 0) + 1) * C
    acc = acc_ref[...]
    argn = argn_ref[...]
    for s in range(BC // FW):
        y_s = jax.lax.slice(y, (0, s * FW), (R, (s + 1) * FW))
        n_s = jax.lax.slice(n, (0, s * FW), (R, (s + 1) * FW))
        upd = (y_s > acc) & (n_s < rowlim)
        acc = jnp.where(upd, y_s, acc)
        argn = jnp.where(upd, n_s, argn)
    acc_ref[...] = acc
    argn_ref[...] = argn

    @pl.when(step < GRID - 1)
    def _():
        o_ref[...] = jnp.zeros(shape, jnp.float32)

    @pl.when(step == GRID - 1)
    def _():
        m = jnp.max(acc, axis=1, keepdims=True)
        candn = jnp.where(acc == m, argn, jnp.int32(2**31 - 1))
        an = jnp.min(candn, axis=1, keepdims=True)
        riota = jax.lax.broadcasted_iota(jnp.int32, (R, 1), 0)
        argf_ref[...] = an - riota * C
        o_ref[...] = (n == an).astype(jnp.float32)


_LAST_BASE = (GRID - 1) * BC
_MAX_WIN = _LAST_BASE - 128


def _patch_body(argc_v_ref, argc_s_ref, z_ref, o_ref, pat_ref, sem):
    copies = []
    for r in range(R):
        g = r // 8
        base = jnp.minimum((argc_s_ref[r, 0] // 128) * 128, _MAX_WIN)
        argc_g = argc_v_ref[pl.ds(8 * g, 8), :]
        lanes = jax.lax.broadcasted_iota(jnp.int32, (8, 128), 1) + base
        pat_ref[r] = (argc_g == lanes).astype(jnp.float32)
        cp = pltpu.make_async_copy(
            pat_ref.at[r],
            o_ref.at[pl.ds(8 * g, 8), pl.ds(base, 128)],
            sem)
        cp.start()
        copies.append(cp)
    for cp in copies:
        cp.wait()


def _patch_ones(argc, zeros_out):
    return pl.pallas_call(
        _patch_body,
        in_specs=[
            pl.BlockSpec(memory_space=pltpu.VMEM),
            pl.BlockSpec(memory_space=pltpu.SMEM),
            pl.BlockSpec(memory_space=pl.ANY),
        ],
        out_specs=pl.BlockSpec(memory_space=pl.ANY),
        out_shape=jax.ShapeDtypeStruct((R, C), jnp.float32),
        scratch_shapes=[
            pltpu.VMEM((R, 8, 128), jnp.float32),
            pltpu.SemaphoreType.DMA,
        ],
        input_output_aliases={2: 0},
    )(argc, argc, zeros_out)


@jax.jit
def kernel(logits):
    zeros_out, argf = pl.pallas_call(
        _argmax_body,
        grid=(GRID,),
        in_specs=[pl.BlockSpec((R, BC), lambda i: (0, i))],
        out_specs=[
            pl.BlockSpec((R, BC), lambda i: (0, i)),
            pl.BlockSpec((R, 1), lambda i: (0, 0)),
        ],
        out_shape=[
            jax.ShapeDtypeStruct((R, C), jnp.float32),
            jax.ShapeDtypeStruct((R, 1), jnp.int32),
        ],
        scratch_shapes=[
            pltpu.VMEM((R, BC), jnp.int32),
            pltpu.VMEM((R, FW), jnp.float32),
            pltpu.VMEM((R, FW), jnp.int32),
        ],
        compiler_params=pltpu.CompilerParams(
            dimension_semantics=("arbitrary",)),
    )(logits)
    return _patch_ones(argf, zeros_out)


# speedup vs baseline: 1.7137x; 1.0120x over previous
import jax
import jax.numpy as jnp
from jax.experimental import pallas as pl
from jax.experimental.pallas import tpu as pltpu

R, C = 64, 100000
BC = 2048
GRID = (C + BC - 1) // BC
FW = 256


def _rotl(x, r):
    return (x << jnp.uint32(r)) | (x >> jnp.uint32(32 - r))


def _threefry_bits(n):
    k0 = jnp.uint32(0)
    k1 = jnp.uint32(42)
    ks = [k0, k1, jnp.uint32(0x1BD11BDA) ^ k0 ^ k1]
    rot_even = (13, 15, 26, 6)
    rot_odd = (17, 29, 16, 24)
    t = n + k1
    x0 = t
    x1 = _rotl(t, 13) ^ t
    for r in rot_even[1:]:
        x0 = x0 + x1
        x1 = _rotl(x1, r)
        x1 = x1 ^ x0
    x0 = x0 + ks[1]
    x1 = x1 + ks[2] + jnp.uint32(1)
    for i in range(1, 5):
        for r in rot_even if i % 2 == 0 else rot_odd:
            x0 = x0 + x1
            x1 = _rotl(x1, r)
            x1 = x1 ^ x0
        x0 = x0 + ks[(i + 1) % 3]
        x1 = x1 + ks[(i + 2) % 3] + jnp.uint32(i + 1)
    return x0 ^ x1


def _gumbel(n):
    bits = _threefry_bits(n.astype(jnp.uint32))
    fb = (bits >> jnp.uint32(9)) | jnp.uint32(0x3F800000)
    u = jax.lax.bitcast_convert_type(fb, jnp.float32) - jnp.float32(1.0)
    eps = jnp.float32(1e-10)
    return -jnp.log(-jnp.log(u + eps) + eps)


def _argmax_body(x_ref, o_ref, argf_ref, n_ref, acc_ref, argn_ref):
    step = pl.program_id(0)
    shape = (R, BC)

    @pl.when(step == 0)
    def _():
        cols = jax.lax.broadcasted_iota(jnp.int32, shape, 1)
        rows = jax.lax.broadcasted_iota(jnp.int32, shape, 0)
        n_ref[...] = rows * C + cols
        acc_ref[...] = jnp.full((R, FW), -jnp.inf, jnp.float32)
        argn_ref[...] = jnp.zeros((R, FW), jnp.int32)

    n = n_ref[...]
    y = x_ref[...] + _gumbel(n)
    n_ref[...] = n + BC

    rowlim = (jax.lax.broadcasted_iota(jnp.int32, (R, 1), 0) + 1) * C
    nsub = BC // FW
    leaves = []
    for s in range(nsub):
        y_s = jax.lax.slice(y, (0, s * FW), (R, (s + 1) * FW))
        n_s = jax.lax.slice(n, (0, s * FW), (R, (s + 1) * FW))
        if s >= nsub - 2:
            y_s = jnp.where(n_s < rowlim, y_s, -jnp.inf)
        leaves.append((y_s, n_s))

    def merge(a, b):
        upd = b[0] > a[0]
        return jnp.where(upd, b[0], a[0]), jnp.where(upd, b[1], a[1])

    while len(leaves) > 1:
        leaves = [merge(leaves[i], leaves[i + 1])
                  for i in range(0, len(leaves), 2)]
    acc, argn = merge((acc_ref[...], argn_ref[...]), leaves[0])
    acc_ref[...] = acc
    argn_ref[...] = argn

    @pl.when(step < GRID - 1)
    def _():
        o_ref[...] = jnp.zeros(shape, jnp.float32)

    @pl.when(step == GRID - 1)
    def _():
        m = jnp.max(acc, axis=1, keepdims=True)
        candn = jnp.where(acc == m, argn, jnp.int32(2**31 - 1))
        an = jnp.min(candn, axis=1, keepdims=True)
        riota = jax.lax.broadcasted_iota(jnp.int32, (R, 1), 0)
        argf_ref[...] = an - riota * C
        o_ref[...] = (n == an).astype(jnp.float32)


_LAST_BASE = (GRID - 1) * BC
_MAX_WIN = _LAST_BASE - 128


def _patch_body(argc_v_ref, argc_s_ref, z_ref, o_ref, pat_ref, sem):
    copies = []
    for r in range(R):
        g = r // 8
        base = jnp.minimum((argc_s_ref[r, 0] // 128) * 128, _MAX_WIN)
        argc_g = argc_v_ref[pl.ds(8 * g, 8), :]
        lanes = jax.lax.broadcasted_iota(jnp.int32, (8, 128), 1) + base
        pat_ref[r] = (argc_g == lanes).astype(jnp.float32)
        cp = pltpu.make_async_copy(
            pat_ref.at[r],
            o_ref.at[pl.ds(8 * g, 8), pl.ds(base, 128)],
            sem)
        cp.start()
        copies.append(cp)
    for cp in copies:
        cp.wait()


def _patch_ones(argc, zeros_out):
    return pl.pallas_call(
        _patch_body,
        in_specs=[
            pl.BlockSpec(memory_space=pltpu.VMEM),
            pl.BlockSpec(memory_space=pltpu.SMEM),
            pl.BlockSpec(memory_space=pl.ANY),
        ],
        out_specs=pl.BlockSpec(memory_space=pl.ANY),
        out_shape=jax.ShapeDtypeStruct((R, C), jnp.float32),
        scratch_shapes=[
            pltpu.VMEM((R, 8, 128), jnp.float32),
            pltpu.SemaphoreType.DMA,
        ],
        input_output_aliases={2: 0},
    )(argc, argc, zeros_out)


@jax.jit
def kernel(logits):
    zeros_out, argf = pl.pallas_call(
        _argmax_body,
        grid=(GRID,),
        in_specs=[pl.BlockSpec((R, BC), lambda i: (0, i))],
        out_specs=[
            pl.BlockSpec((R, BC), lambda i: (0, i)),
            pl.BlockSpec((R, 1), lambda i: (0, 0)),
        ],
        out_shape=[
            jax.ShapeDtypeStruct((R, C), jnp.float32),
            jax.ShapeDtypeStruct((R, 1), jnp.int32),
        ],
        scratch_shapes=[
            pltpu.VMEM((R, BC), jnp.int32),
            pltpu.VMEM((R, FW), jnp.float32),
            pltpu.VMEM((R, FW), jnp.int32),
        ],
        compiler_params=pltpu.CompilerParams(
            dimension_semantics=("arbitrary",)),
    )(logits)
    return _patch_ones(argf, zeros_out)
